# baseline (speedup 1.0000x reference)
import functools

import jax
import jax.numpy as jnp
from jax import lax
from jax.experimental import pallas as pl
from jax.experimental.pallas import tpu as pltpu
from jax.experimental.pallas import tpu_sc as plsc

NUM_EMB = 1024
DIM = 64
COMMIT = 0.25

SC_CORES = 2
SC_SUBCORES = 16
SC_WORKERS = SC_CORES * SC_SUBCORES


def _vq_argmin_kernel(x_ref, e_ref, iota_ref, idx_ref, loss_ref):
    xp = x_ref[0]
    e = e_ref[...]
    iota_f = iota_ref[...]
    p = xp.shape[0]

    x = jnp.transpose(xp, (1, 0))

    x_sq = jnp.sum(x * x, axis=0, keepdims=True)
    e_sq = jnp.sum(e * e, axis=1, keepdims=True)
    s2 = jax.lax.dot_general(
        e + e, x, (((1,), (0,)), ((), ())),
        preferred_element_type=jnp.float32)
    dist = (x_sq + e_sq) - s2

    m = jnp.min(dist, axis=0, keepdims=True)
    idx_f = jnp.min(
        jnp.where(dist == m, iota_f, jnp.float32(NUM_EMB)),
        axis=0, keepdims=True)

    loss_ref[pl.program_id(0)] = jnp.sum(m) * ((1.0 + COMMIT) / (DIM * p))
    idx_ref[0] = idx_f.astype(jnp.int32)


def _sc_gather(table, idx_flat):
    n = idx_flat.shape[0]
    per_w = n // SC_WORKERS
    table_pad = jnp.pad(table, ((0, 0), (0, 128 - DIM)))
    mesh = plsc.VectorSubcoreMesh(
        core_axis_name="c", subcore_axis_name="s")

    @functools.partial(
        pl.kernel,
        out_type=jax.ShapeDtypeStruct((n, 128), jnp.float32),
        mesh=mesh,
        scratch_types=[
            pltpu.VMEM((per_w,), jnp.int32),
            pltpu.VMEM((per_w, 128), jnp.float32),
            pltpu.SemaphoreType.DMA,
        ],
    )
    def gather_kernel(table_hbm, idx_hbm, out_hbm, idx_v, rows_v, sem):
        wid = lax.axis_index("s") * SC_CORES + lax.axis_index("c")
        base = wid * per_w
        pltpu.sync_copy(idx_hbm.at[pl.ds(base, per_w)], idx_v)
        pltpu.async_copy(table_hbm.at[idx_v], rows_v, sem).wait()
        pltpu.sync_copy(rows_v, out_hbm.at[pl.ds(base, per_w)])

    return gather_kernel(table_pad, idx_flat)[:, :DIM]


@functools.partial(jax.jit, static_argnames=())
def kernel(inputs, embedding):
    b, c, h, w = inputs.shape
    p = h * w
    x = jnp.transpose(inputs, (0, 2, 3, 1)).reshape(b, p, c)
    iota_f = jax.lax.broadcasted_iota(jnp.float32, (NUM_EMB, p), 0)

    idx, loss = pl.pallas_call(
        _vq_argmin_kernel,
        grid=(b,),
        in_specs=[
            pl.BlockSpec((1, p, c), lambda i: (i, 0, 0)),
            pl.BlockSpec((NUM_EMB, DIM), lambda i: (0, 0)),
            pl.BlockSpec((NUM_EMB, p), lambda i: (0, 0)),
        ],
        out_specs=[
            pl.BlockSpec((1, 1, p), lambda i: (i, 0, 0)),
            pl.BlockSpec((b,), lambda i: (0,),
                         memory_space=pltpu.SMEM),
        ],
        out_shape=[
            jax.ShapeDtypeStruct((b, 1, p), jnp.int32),
            jax.ShapeDtypeStruct((b,), jnp.float32),
        ],
        compiler_params=pltpu.CompilerParams(
            dimension_semantics=("parallel",)),
    )(x, embedding, iota_f)

    q = _sc_gather(embedding, idx.reshape(b * p))

    quantized_st = jnp.transpose(q.reshape(b, h, w, c), (0, 3, 1, 2))
    enc_idx = idx.reshape(b, h, w)
    return (quantized_st, enc_idx, loss)


# speedup vs baseline: 1.4978x; 1.4978x over previous
import functools

import jax
import jax.numpy as jnp
from jax.experimental import pallas as pl
from jax.experimental.pallas import tpu as pltpu

NUM_EMB = 1024
DIM = 64
COMMIT = 0.25


def _vq_kernel(x_ref, e_ref, q_ref, idx_ref, loss_ref):
    xp = x_ref[0]
    e = e_ref[...]
    p = xp.shape[0]

    x = jnp.transpose(xp, (1, 0))

    x_sq = jnp.sum(x * x, axis=0, keepdims=True)
    e_sq = jnp.sum(e * e, axis=1, keepdims=True)
    s2 = jax.lax.dot_general(
        e + e, x, (((1,), (0,)), ((), ())),
        preferred_element_type=jnp.float32)
    dist = (x_sq + e_sq) - s2

    m = jnp.min(dist, axis=0, keepdims=True)
    iota_j = jax.lax.broadcasted_iota(jnp.int32, dist.shape, 0)
    idx = jnp.min(jnp.where(dist == m, iota_j, jnp.int32(NUM_EMB)),
                  axis=0, keepdims=True)

    onehot = (iota_j == idx).astype(jnp.float32)
    q = jax.lax.dot_general(
        e, onehot, (((0,), (0,)), ((), ())),
        preferred_element_type=jnp.float32)

    loss = jnp.sum(m) * ((1.0 + COMMIT) / (DIM * p))

    q_ref[0] = jnp.transpose(q, (1, 0))
    idx_ref[0] = idx
    loss_ref[0] = jnp.full((1, 128), loss, dtype=jnp.float32)


@functools.partial(jax.jit, static_argnames=())
def kernel(inputs, embedding):
    b, c, h, w = inputs.shape
    p = h * w
    x = jnp.transpose(inputs, (0, 2, 3, 1)).reshape(b, p, c)

    q, idx, loss = pl.pallas_call(
        _vq_kernel,
        grid=(b,),
        in_specs=[
            pl.BlockSpec((1, p, c), lambda i: (i, 0, 0)),
            pl.BlockSpec((NUM_EMB, DIM), lambda i: (0, 0)),
        ],
        out_specs=[
            pl.BlockSpec((1, p, c), lambda i: (i, 0, 0)),
            pl.BlockSpec((1, 1, p), lambda i: (i, 0, 0)),
            pl.BlockSpec((1, 1, 128), lambda i: (i, 0, 0)),
        ],
        out_shape=[
            jax.ShapeDtypeStruct((b, p, c), jnp.float32),
            jax.ShapeDtypeStruct((b, 1, p), jnp.int32),
            jax.ShapeDtypeStruct((b, 1, 128), jnp.float32),
        ],
        compiler_params=pltpu.CompilerParams(
            dimension_semantics=("parallel",)),
    )(x, embedding)

    quantized_st = jnp.transpose(q.reshape(b, h, w, c), (0, 3, 1, 2))
    enc_idx = idx.reshape(b, h, w)
    loss_out = loss[:, 0, 0]
    return (quantized_st, enc_idx, loss_out)
